# Initial kernel scaffold; baseline (speedup 1.0000x reference)
import jax
import jax.numpy as jnp
from jax.experimental import pallas as pl


def kernel(x, edge_index, W_l, b_l, W_r, b_r, att, bias_gat, W1, b1, W2, b2):
    raise NotImplementedError("write your pallas kernel here")


# speedup vs baseline: 91.7609x; 91.7609x over previous
import functools

import jax
import jax.numpy as jnp
from jax import lax
from jax.experimental import pallas as pl
from jax.experimental.pallas import tpu as pltpu
from jax.experimental.pallas import tpu_sc as plsc

N = 10000
E = 320000
D = 128
H = 2
C = 16
HC = H * C

NC = 2
NS = 16
NW = NC * NS
EPW = E // NW
CH = 1000
NCHUNK = EPW // CH
SUB = 125
NSUB = CH // SUB
RPT = N // NS
RB = 2000

@functools.cache
def _mesh():
    return plsc.VectorSubcoreMesh(core_axis_name="c", subcore_axis_name="s",
                                  num_cores=NC, num_subcores=NS)


def _xform_body(x_ref, wl_ref, bl_ref, wr_ref, br_ref, xl_ref, xr_ref):
    xb = x_ref[...]
    xl_ref[...] = jnp.dot(xb, wl_ref[...], preferred_element_type=jnp.float32) + bl_ref[...]
    xr_ref[...] = jnp.dot(xb, wr_ref[...], preferred_element_type=jnp.float32) + br_ref[...]


def _xform(x, W_l, b_l, W_r, b_r):
    return pl.pallas_call(
        _xform_body,
        grid=(N // RB,),
        in_specs=[
            pl.BlockSpec((RB, D), lambda i: (i, 0)),
            pl.BlockSpec((D, HC), lambda i: (0, 0)),
            pl.BlockSpec((1, HC), lambda i: (0, 0)),
            pl.BlockSpec((D, HC), lambda i: (0, 0)),
            pl.BlockSpec((1, HC), lambda i: (0, 0)),
        ],
        out_specs=[
            pl.BlockSpec((RB, HC), lambda i: (i, 0)),
            pl.BlockSpec((RB, HC), lambda i: (i, 0)),
        ],
        out_shape=[
            jax.ShapeDtypeStruct((N, HC), jnp.float32),
            jax.ShapeDtypeStruct((N, HC), jnp.float32),
        ],
    )(x, W_l, b_l, W_r, b_r)


@functools.cache
def _gather_kernel():
    return functools.partial(
        pl.kernel,
        out_type=[
            jax.ShapeDtypeStruct((E, HC), jnp.float32),
            jax.ShapeDtypeStruct((E, HC), jnp.float32),
        ],
        mesh=_mesh(),
        compiler_params=pltpu.CompilerParams(use_tc_tiling_on_sc=False),
        scratch_types=[
            pltpu.VMEM((NSUB, SUB), jnp.int32),
            pltpu.VMEM((NSUB, SUB), jnp.int32),
            pltpu.VMEM((CH, HC), jnp.float32),
            pltpu.VMEM((CH, HC), jnp.float32),
            pltpu.SemaphoreType.DMA,
        ],
    )(_gather_body)


def _gather_body(xl_hbm, xr_hbm, src2_hbm, dst2_hbm, xj_hbm, xi_hbm,
                 src_v, dst_v, rl_v, ri_v, sem):
    wid = lax.axis_index("c") * NS + lax.axis_index("s")

    def chunk(ci, carry):
        base = wid * EPW + ci * CH
        rowbase = wid * (EPW // SUB) + ci * NSUB
        pltpu.sync_copy(src2_hbm.at[pl.ds(rowbase, NSUB)], src_v)
        pltpu.sync_copy(dst2_hbm.at[pl.ds(rowbase, NSUB)], dst_v)
        descs = []
        for j in range(NSUB):
            sl = pl.ds(j * SUB, SUB)
            descs.append(pltpu.async_copy(xl_hbm.at[src_v.at[j]], rl_v.at[sl], sem))
            descs.append(pltpu.async_copy(xr_hbm.at[dst_v.at[j]], ri_v.at[sl], sem))
        for dsc in descs:
            dsc.wait()
        pltpu.sync_copy(rl_v, xj_hbm.at[pl.ds(base, CH)])
        pltpu.sync_copy(ri_v, xi_hbm.at[pl.ds(base, CH)])
        return carry

    lax.fori_loop(0, NCHUNK, chunk, 0)


def _edge_body(xj_ref, xi_ref, a2p_ref, bb_ref, w_ref, p_ref):
    xj = xj_ref[...]
    u = xj + xi_ref[...]
    s = jnp.maximum(u, 0.2 * u)
    logit8 = jnp.dot(s, a2p_ref[...], preferred_element_type=jnp.float32)
    p8 = jnp.exp(logit8)
    pE = jnp.dot(p8, bb_ref[...], preferred_element_type=jnp.float32)
    w_ref[...] = xj * pE
    p_ref[...] = p8


def _edge(xj4, xi4, A2p, BB):
    E4 = E // 4
    return pl.pallas_call(
        _edge_body,
        grid=(E4 // RB,),
        in_specs=[
            pl.BlockSpec((RB, 128), lambda i: (i, 0)),
            pl.BlockSpec((RB, 128), lambda i: (i, 0)),
            pl.BlockSpec((128, 8), lambda i: (0, 0)),
            pl.BlockSpec((8, 128), lambda i: (0, 0)),
        ],
        out_specs=[
            pl.BlockSpec((RB, 128), lambda i: (i, 0)),
            pl.BlockSpec((RB, 8), lambda i: (i, 0)),
        ],
        out_shape=[
            jax.ShapeDtypeStruct((E4, 128), jnp.float32),
            jax.ShapeDtypeStruct((E4, 8), jnp.float32),
        ],
    )(xj4, xi4, A2p, BB)


DW = 2048
DR = 16


@functools.cache
def _scatter_kernel():
    return functools.partial(
        pl.kernel,
        out_type=[
            jax.ShapeDtypeStruct((NC * N, HC), jnp.float32),
            jax.ShapeDtypeStruct((NC * DR, DW), jnp.float32),
        ],
        mesh=_mesh(),
        compiler_params=pltpu.CompilerParams(use_tc_tiling_on_sc=False,
                                             needs_layout_passes=False),
        scratch_types=[
            pltpu.VMEM((NSUB, SUB), jnp.int32),
            pltpu.VMEM((CH,), jnp.int32),
            pltpu.VMEM((CH, HC), jnp.float32),
            pltpu.VMEM((2 * CH,), jnp.float32),
            pltpu.VMEM((DR, DW), jnp.float32),
            pltpu.VMEM((16,), jnp.int32),
            pltpu.VMEM_SHARED((N, HC), jnp.float32),
            pltpu.VMEM_SHARED((DR, DW), jnp.float32),
        ],
    )(_scatter_body)


def _scatter_body(w_hbm, p_hbm, dst2_hbm, dstf_hbm, z32_hbm, zden_hbm,
                  num_out, den_out, dst2_v, dstf_v, w_v, p_v, den_v, idx16_v,
                  accum, den_sh):
    cid = lax.axis_index("c")
    sid = lax.axis_index("s")
    wid = cid * NS + sid
    iota16 = lax.iota(jnp.int32, 16)
    idx16_v[...] = iota16
    pltpu.sync_copy(zden_hbm, den_v)
    pltpu.sync_copy(z32_hbm.at[pl.ds(sid * RPT, RPT)], accum.at[pl.ds(sid * RPT, RPT)])

    @pl.when(sid == 0)
    def _():
        pltpu.sync_copy(zden_hbm, den_sh)

    plsc.subcore_barrier()

    def chunk(ci, carry):
        base = wid * EPW + ci * CH
        pltpu.sync_copy(w_hbm.at[pl.ds(base, CH)], w_v)
        pltpu.sync_copy(dst2_hbm.at[pl.ds(wid * (EPW // SUB) + ci * NSUB, NSUB)], dst2_v)
        pltpu.sync_copy(dstf_hbm.at[pl.ds(base, CH)], dstf_v)
        pltpu.sync_copy(p_hbm.at[pl.ds(2 * base, 2 * CH)], p_v)
        for j in range(NSUB):
            pltpu.sync_copy(w_v.at[pl.ds(j * SUB, SUB)], accum.at[dst2_v.at[j]], add=True)

        def dgroup(g, c2):
            pv = p_v[pl.ds(g * 16, 16)]
            ec = g * 8 + (iota16 >> 1)
            dvals = plsc.load_gather(dstf_v, [ec])
            sidx = dvals * 2 + (iota16 & 1)
            plsc.addupdate_scatter(den_v, [sidx >> 11, sidx & (DW - 1)], pv)
            return c2

        lax.fori_loop(0, 2 * CH // 16, dgroup, 0)
        return carry

    lax.fori_loop(0, NCHUNK, chunk, 0)
    pltpu.sync_copy(den_v, den_sh.at[idx16_v], add=True)
    plsc.subcore_barrier()
    pltpu.sync_copy(accum.at[pl.ds(sid * RPT, RPT)],
                    num_out.at[pl.ds(cid * N + sid * RPT, RPT)])

    @pl.when(sid == 0)
    def _():
        pltpu.sync_copy(den_sh, den_out.at[pl.ds(cid * DR, DR)])


def _final_body(np_ref, dp_ref, b2b_ref, bias_ref, w1_ref, b1_ref, w2_ref,
                b2_ref, q_ref):
    num = np_ref[0] + np_ref[1]
    den = jnp.sum(dp_ref[...], axis=0)
    recip = 1.0 / (den + 1e-16)
    recip_b = jnp.dot(recip, b2b_ref[...], preferred_element_type=jnp.float32)
    outg = num * recip_b + bias_ref[...]
    hmid = jnp.maximum(
        jnp.dot(outg, w1_ref[...], preferred_element_type=jnp.float32) + b1_ref[...], 0.0)
    q_ref[...] = jnp.dot(hmid, w2_ref[...], preferred_element_type=jnp.float32) + b2_ref[...]


def _final(num_part, den_part, B2b, bias_gat, W1, b1, W2, b2):
    return pl.pallas_call(
        _final_body,
        grid=(N // RB,),
        in_specs=[
            pl.BlockSpec((NC, RB, HC), lambda i: (0, i, 0)),
            pl.BlockSpec((NC, RB, 2), lambda i: (0, i, 0)),
            pl.BlockSpec((2, HC), lambda i: (0, 0)),
            pl.BlockSpec((1, HC), lambda i: (0, 0)),
            pl.BlockSpec((HC, 32), lambda i: (0, 0)),
            pl.BlockSpec((1, 32), lambda i: (0, 0)),
            pl.BlockSpec((32, 2), lambda i: (0, 0)),
            pl.BlockSpec((1, 2), lambda i: (0, 0)),
        ],
        out_specs=pl.BlockSpec((RB, 2), lambda i: (i, 0)),
        out_shape=jax.ShapeDtypeStruct((N, 2), jnp.float32),
    )(num_part, den_part, B2b, bias_gat, W1, b1, W2, b2)


def kernel(x, edge_index, W_l, b_l, W_r, b_r, att, bias_gat, W1, b1, W2, b2):
    src = edge_index[0]
    dst = edge_index[1]

    xl, xr = _xform(x, W_l, b_l.reshape(1, HC), W_r, b_r.reshape(1, HC))
    src2 = src.reshape(E // SUB, SUB)
    dst2 = dst.reshape(E // SUB, SUB)
    xj, xi = _gather_kernel()(xl, xr, src2, dst2)

    kk = jnp.arange(128)
    mm = jnp.arange(8)
    attf = jnp.tile(att.reshape(HC), 4)
    A2p = attf[:, None] * ((kk[:, None] // C) == mm[None, :]).astype(jnp.float32)
    BB = (mm[:, None] == (kk[None, :] // C)).astype(jnp.float32)

    w4, p8 = _edge(xj.reshape(E // 4, 128), xi.reshape(E // 4, 128), A2p, BB)
    w = w4.reshape(E, HC)
    pflat = p8.reshape(2 * E)

    z32 = jnp.zeros((N, HC), jnp.float32)
    zden = jnp.zeros((DR, DW), jnp.float32)
    num_part, den_part = _scatter_kernel()(w, pflat, dst2, dst, z32, zden)
    den3 = den_part.reshape(NC, DR * DW)[:, :2 * N].reshape(NC, N, 2)

    hh = jnp.arange(2)
    B2b = (hh[:, None] == (jnp.arange(HC)[None, :] // C)).astype(jnp.float32)
    q = _final(num_part.reshape(NC, N, HC), den3,
               B2b, bias_gat.reshape(1, HC), W1, b1.reshape(1, 32),
               W2, b2.reshape(1, 2))
    return q


# speedup vs baseline: 96.3506x; 1.0500x over previous
import functools

import jax
import jax.numpy as jnp
from jax import lax
from jax.experimental import pallas as pl
from jax.experimental.pallas import tpu as pltpu
from jax.experimental.pallas import tpu_sc as plsc

N = 10000
E = 320000
D = 128
H = 2
C = 16
HC = H * C

NC = 2
NS = 16
NW = NC * NS
EPW = E // NW
CH = 1000
NCHUNK = EPW // CH
SUB = 125
NSUB = CH // SUB
RPT = N // NS
RB = 2000

@functools.cache
def _mesh():
    return plsc.VectorSubcoreMesh(core_axis_name="c", subcore_axis_name="s",
                                  num_cores=NC, num_subcores=NS)


def _xform_body(x_ref, wl_ref, bl_ref, wr_ref, br_ref, xl_ref, xr_ref):
    xb = x_ref[...]
    xl_ref[...] = jnp.dot(xb, wl_ref[...], preferred_element_type=jnp.float32) + bl_ref[...]
    xr_ref[...] = jnp.dot(xb, wr_ref[...], preferred_element_type=jnp.float32) + br_ref[...]


def _xform(x, W_l, b_l, W_r, b_r):
    return pl.pallas_call(
        _xform_body,
        grid=(N // RB,),
        in_specs=[
            pl.BlockSpec((RB, D), lambda i: (i, 0)),
            pl.BlockSpec((D, HC), lambda i: (0, 0)),
            pl.BlockSpec((1, HC), lambda i: (0, 0)),
            pl.BlockSpec((D, HC), lambda i: (0, 0)),
            pl.BlockSpec((1, HC), lambda i: (0, 0)),
        ],
        out_specs=[
            pl.BlockSpec((RB, HC), lambda i: (i, 0)),
            pl.BlockSpec((RB, HC), lambda i: (i, 0)),
        ],
        out_shape=[
            jax.ShapeDtypeStruct((N, HC), jnp.float32),
            jax.ShapeDtypeStruct((N, HC), jnp.float32),
        ],
    )(x, W_l, b_l, W_r, b_r)


@functools.cache
def _gather_kernel():
    return functools.partial(
        pl.kernel,
        out_type=[
            jax.ShapeDtypeStruct((E, HC), jnp.float32),
            jax.ShapeDtypeStruct((E, HC), jnp.float32),
        ],
        mesh=_mesh(),
        compiler_params=pltpu.CompilerParams(use_tc_tiling_on_sc=False),
        scratch_types=[
            pltpu.VMEM((NSUB, SUB), jnp.int32),
            pltpu.VMEM((NSUB, SUB), jnp.int32),
            pltpu.VMEM((CH, HC), jnp.float32),
            pltpu.VMEM((CH, HC), jnp.float32),
            pltpu.SemaphoreType.DMA,
        ],
    )(_gather_body)


def _gather_body(xl_hbm, xr_hbm, src2_hbm, dst2_hbm, xj_hbm, xi_hbm,
                 src_v, dst_v, rl_v, ri_v, sem):
    wid = lax.axis_index("c") * NS + lax.axis_index("s")

    def chunk(ci, carry):
        base = wid * EPW + ci * CH
        rowbase = wid * (EPW // SUB) + ci * NSUB
        pltpu.sync_copy(src2_hbm.at[pl.ds(rowbase, NSUB)], src_v)
        pltpu.sync_copy(dst2_hbm.at[pl.ds(rowbase, NSUB)], dst_v)
        descs = []
        for j in range(NSUB):
            sl = pl.ds(j * SUB, SUB)
            descs.append(pltpu.async_copy(xl_hbm.at[src_v.at[j]], rl_v.at[sl], sem))
            descs.append(pltpu.async_copy(xr_hbm.at[dst_v.at[j]], ri_v.at[sl], sem))
        for dsc in descs:
            dsc.wait()
        pltpu.sync_copy(rl_v, xj_hbm.at[pl.ds(base, CH)])
        pltpu.sync_copy(ri_v, xi_hbm.at[pl.ds(base, CH)])
        return carry

    lax.fori_loop(0, NCHUNK, chunk, 0)


def _edge_body(xj_ref, xi_ref, a2p_ref, bb_ref, w_ref, p_ref):
    xj = xj_ref[...]
    u = xj + xi_ref[...]
    s = jnp.maximum(u, 0.2 * u)
    logit8 = jnp.dot(s, a2p_ref[...], preferred_element_type=jnp.float32)
    p8 = jnp.exp(logit8)
    pE = jnp.dot(p8, bb_ref[...], preferred_element_type=jnp.float32)
    w_ref[...] = xj * pE
    p_ref[...] = p8


def _edge(xj4, xi4, A2p, BB):
    E4 = E // 4
    return pl.pallas_call(
        _edge_body,
        grid=(E4 // RB,),
        in_specs=[
            pl.BlockSpec((RB, 128), lambda i: (i, 0)),
            pl.BlockSpec((RB, 128), lambda i: (i, 0)),
            pl.BlockSpec((128, 8), lambda i: (0, 0)),
            pl.BlockSpec((8, 128), lambda i: (0, 0)),
        ],
        out_specs=[
            pl.BlockSpec((RB, 128), lambda i: (i, 0)),
            pl.BlockSpec((RB, 8), lambda i: (i, 0)),
        ],
        out_shape=[
            jax.ShapeDtypeStruct((E4, 128), jnp.float32),
            jax.ShapeDtypeStruct((E4, 8), jnp.float32),
        ],
    )(xj4, xi4, A2p, BB)


DW = 2048
DR = 16


@functools.cache
def _scatter_kernel():
    return functools.partial(
        pl.kernel,
        out_type=[
            jax.ShapeDtypeStruct((NC * N, HC), jnp.float32),
            jax.ShapeDtypeStruct((NC * DR, DW), jnp.float32),
        ],
        mesh=_mesh(),
        compiler_params=pltpu.CompilerParams(use_tc_tiling_on_sc=False,
                                             needs_layout_passes=False),
        scratch_types=[
            pltpu.VMEM((NSUB, SUB), jnp.int32),
            pltpu.VMEM((CH,), jnp.int32),
            pltpu.VMEM((CH, HC), jnp.float32),
            pltpu.VMEM((2 * CH,), jnp.float32),
            pltpu.VMEM((DR, DW), jnp.float32),
            pltpu.VMEM((16,), jnp.int32),
            pltpu.VMEM_SHARED((N, HC), jnp.float32),
            pltpu.VMEM_SHARED((DR, DW), jnp.float32),
            pltpu.SemaphoreType.DMA,
        ],
    )(_scatter_body)


def _scatter_body(w_hbm, p_hbm, dst2_hbm, dstf_hbm, z32_hbm, zden_hbm,
                  num_out, den_out, dst2_v, dstf_v, w_v, p_v, den_v, idx16_v,
                  accum, den_sh, sem):
    cid = lax.axis_index("c")
    sid = lax.axis_index("s")
    wid = cid * NS + sid
    iota16 = lax.iota(jnp.int32, 16)
    idx16_v[...] = iota16
    pltpu.sync_copy(zden_hbm, den_v)
    pltpu.sync_copy(z32_hbm.at[pl.ds(sid * RPT, RPT)], accum.at[pl.ds(sid * RPT, RPT)])

    @pl.when(sid == 0)
    def _():
        pltpu.sync_copy(zden_hbm, den_sh)

    plsc.subcore_barrier()

    def chunk(ci, carry):
        base = wid * EPW + ci * CH
        pltpu.sync_copy(w_hbm.at[pl.ds(base, CH)], w_v)
        pltpu.sync_copy(dst2_hbm.at[pl.ds(wid * (EPW // SUB) + ci * NSUB, NSUB)], dst2_v)
        pltpu.sync_copy(dstf_hbm.at[pl.ds(base, CH)], dstf_v)
        pltpu.sync_copy(p_hbm.at[pl.ds(2 * base, 2 * CH)], p_v)
        descs = []
        for j in range(NSUB):
            descs.append(pltpu.async_copy(
                w_v.at[pl.ds(j * SUB, SUB)], accum.at[dst2_v.at[j]], sem, add=True))

        def dgroup(g, c2):
            pv = p_v[pl.ds(g * 16, 16)]
            ec = g * 8 + (iota16 >> 1)
            dvals = plsc.load_gather(dstf_v, [ec])
            sidx = dvals * 2 + (iota16 & 1)
            plsc.addupdate_scatter(den_v, [sidx >> 11, sidx & (DW - 1)], pv)
            return c2

        lax.fori_loop(0, 2 * CH // 16, dgroup, 0)
        for dsc in descs:
            dsc.wait()
        return carry

    lax.fori_loop(0, NCHUNK, chunk, 0)
    pltpu.sync_copy(den_v, den_sh.at[idx16_v], add=True)
    plsc.subcore_barrier()
    pltpu.sync_copy(accum.at[pl.ds(sid * RPT, RPT)],
                    num_out.at[pl.ds(cid * N + sid * RPT, RPT)])

    @pl.when(sid == 0)
    def _():
        pltpu.sync_copy(den_sh, den_out.at[pl.ds(cid * DR, DR)])


def _final_body(np_ref, dp_ref, b2b_ref, bias_ref, w1_ref, b1_ref, w2_ref,
                b2_ref, q_ref):
    num = np_ref[0] + np_ref[1]
    den = jnp.sum(dp_ref[...], axis=0)
    recip = 1.0 / (den + 1e-16)
    recip_b = jnp.dot(recip, b2b_ref[...], preferred_element_type=jnp.float32)
    outg = num * recip_b + bias_ref[...]
    hmid = jnp.maximum(
        jnp.dot(outg, w1_ref[...], preferred_element_type=jnp.float32) + b1_ref[...], 0.0)
    q_ref[...] = jnp.dot(hmid, w2_ref[...], preferred_element_type=jnp.float32) + b2_ref[...]


def _final(num_part, den_part, B2b, bias_gat, W1, b1, W2, b2):
    return pl.pallas_call(
        _final_body,
        grid=(N // RB,),
        in_specs=[
            pl.BlockSpec((NC, RB, HC), lambda i: (0, i, 0)),
            pl.BlockSpec((NC, RB, 2), lambda i: (0, i, 0)),
            pl.BlockSpec((2, HC), lambda i: (0, 0)),
            pl.BlockSpec((1, HC), lambda i: (0, 0)),
            pl.BlockSpec((HC, 32), lambda i: (0, 0)),
            pl.BlockSpec((1, 32), lambda i: (0, 0)),
            pl.BlockSpec((32, 2), lambda i: (0, 0)),
            pl.BlockSpec((1, 2), lambda i: (0, 0)),
        ],
        out_specs=pl.BlockSpec((RB, 2), lambda i: (i, 0)),
        out_shape=jax.ShapeDtypeStruct((N, 2), jnp.float32),
    )(num_part, den_part, B2b, bias_gat, W1, b1, W2, b2)


def kernel(x, edge_index, W_l, b_l, W_r, b_r, att, bias_gat, W1, b1, W2, b2):
    src = edge_index[0]
    dst = edge_index[1]

    xl, xr = _xform(x, W_l, b_l.reshape(1, HC), W_r, b_r.reshape(1, HC))
    src2 = src.reshape(E // SUB, SUB)
    dst2 = dst.reshape(E // SUB, SUB)
    xj, xi = _gather_kernel()(xl, xr, src2, dst2)

    kk = jnp.arange(128)
    mm = jnp.arange(8)
    attf = jnp.tile(att.reshape(HC), 4)
    A2p = attf[:, None] * ((kk[:, None] // C) == mm[None, :]).astype(jnp.float32)
    BB = (mm[:, None] == (kk[None, :] // C)).astype(jnp.float32)

    w4, p8 = _edge(xj.reshape(E // 4, 128), xi.reshape(E // 4, 128), A2p, BB)
    w = w4.reshape(E, HC)
    pflat = p8.reshape(2 * E)

    z32 = jnp.zeros((N, HC), jnp.float32)
    zden = jnp.zeros((DR, DW), jnp.float32)
    num_part, den_part = _scatter_kernel()(w, pflat, dst2, dst, z32, zden)
    den3 = den_part.reshape(NC, DR * DW)[:, :2 * N].reshape(NC, N, 2)

    hh = jnp.arange(2)
    B2b = (hh[:, None] == (jnp.arange(HC)[None, :] // C)).astype(jnp.float32)
    q = _final(num_part.reshape(NC, N, HC), den3,
               B2b, bias_gat.reshape(1, HC), W1, b1.reshape(1, 32),
               W2, b2.reshape(1, 2))
    return q
